# baseline (speedup 1.0000x reference)
import functools

import jax
import jax.numpy as jnp
from jax import lax
from jax.experimental import pallas as pl
from jax.experimental.pallas import tpu as pltpu
from jax.experimental.pallas import tpu_sc as plsc

N = 10000
E = 320000
D = 128
ED = 16

NC = 2
NS = 16
NW = NC * NS
VS = 64
SE = E // VS
G = 4
PNG = N // G
FB = 8
GCH = 1024
CAPF = 5248

_i32 = jnp.int32
_f32 = jnp.float32


def _iota16():
    return lax.iota(_i32, 16)


def _edge_mm_body(ea_ref, w0_ref, w1_ref, b0_ref, b1_ref):
    ea = ea_ref[...]
    b0_ref[...] = jnp.dot(ea, w0_ref[...], preferred_element_type=_f32)
    b1_ref[...] = jnp.dot(ea, w1_ref[...], preferred_element_type=_f32)


def _edge_mm(ea, we0, we1):
    blk = 2560
    return pl.pallas_call(
        _edge_mm_body,
        grid=(E // blk,),
        in_specs=[
            pl.BlockSpec((blk, ED), lambda i: (i, 0)),
            pl.BlockSpec((ED, D), lambda i: (0, 0)),
            pl.BlockSpec((ED, D), lambda i: (0, 0)),
        ],
        out_specs=[
            pl.BlockSpec((blk, D), lambda i: (i, 0)),
            pl.BlockSpec((blk, D), lambda i: (i, 0)),
        ],
        out_shape=[jax.ShapeDtypeStruct((E, D), _f32)] * 2,
    )(ea, we0, we1)


def _gru_body(emit_hp, x_ref, a_ref, wz, uz, bz, wr, ur, br, wh, uh, bh,
              wg, out_ref):
    x = x_ref[...]
    a = a_ref[...]

    def mm(m, w):
        return jnp.dot(m, w[...], preferred_element_type=_f32)

    z = 1.0 / (1.0 + jnp.exp(-(mm(x, wz) + mm(a, uz) + bz[...])))
    r = 1.0 / (1.0 + jnp.exp(-(mm(x, wr) + mm(a, ur) + br[...])))
    n = jnp.tanh(mm(x, wh) + mm(r * a, uh) + bh[...])
    h = (1.0 - z) * a + z * n
    if emit_hp:
        out_ref[...] = mm(h, wg)
    else:
        out_ref[...] = h


def _gru(x, agg, wz, uz, bz, wr, ur, br, wh, uh, bh, wg=None):
    blk = 2000
    emit_hp = wg is not None
    if wg is None:
        wg = jnp.zeros((D, D), _f32)
    wspec = pl.BlockSpec((D, D), lambda i: (0, 0))
    bspec = pl.BlockSpec((1, D), lambda i: (0, 0))
    return pl.pallas_call(
        functools.partial(_gru_body, emit_hp),
        grid=(N // blk,),
        in_specs=[
            pl.BlockSpec((blk, D), lambda i: (i, 0)),
            pl.BlockSpec((blk, D), lambda i: (i, 0)),
            wspec, wspec, bspec, wspec, wspec, bspec, wspec, wspec, bspec,
            wspec,
        ],
        out_specs=pl.BlockSpec((blk, D), lambda i: (i, 0)),
        out_shape=jax.ShapeDtypeStruct((N, D), _f32),
    )(x, agg, wz, uz, bz, wr, ur, br, wh, uh, bh, wg)


def _part_body(dst_h, src_h, feid, fsrc, fdl, fcnt,
               dbuf, sbuf, stg_e, stg_s, stg_d, cbuf):
    wid = lax.axis_index("s") * NC + lax.axis_index("c")
    iota = _iota16()

    for half in range(2):
        v = wid * 2 + half
        base = v * SE
        pltpu.sync_copy(dst_h.at[pl.ds(base, SE)], dbuf.at[pl.ds(0, SE)])
        pltpu.sync_copy(src_h.at[pl.ds(base, SE)], sbuf.at[pl.ds(0, SE)])
        dbuf[pl.ds(SE, 16)] = jnp.full((16,), 1 << 30, _i32)

        dummy_e = jnp.full((16,), v * 1000, _i32)
        dummy_s = jnp.full((16,), v * 100, _i32)
        dummy_d = jnp.full((16,), PNG * 16, _i32)

        def fill_step(i, _):
            stg_e[pl.ds(i * 16, 16)] = dummy_e
            stg_s[pl.ds(i * 16, 16)] = dummy_s
            stg_d[pl.ds(i * 16, 16)] = dummy_d
            return 0

        lax.fori_loop(0, (G * CAPF) // 16, fill_step, 0)

        def scan_step(i, ps):
            d = dbuf[pl.ds(i * 16, 16)]
            s = sbuf[pl.ds(i * 16, 16)]
            eidv = base + i * 16 + iota
            new_ps = []
            for g in range(G):
                lo = g * PNG
                msk = (d >= lo) & (d < lo + PNG)
                dl16 = (d - lo) << 4
                p = ps[g]
                csum = jnp.cumsum(jnp.where(msk, 1, 0).astype(_i32))
                pos = g * CAPF + p + csum - 1
                plsc.store_scatter(stg_e, [pos], eidv, mask=msk)
                plsc.store_scatter(stg_s, [pos], s, mask=msk)
                plsc.store_scatter(stg_d, [pos], dl16, mask=msk)
                new_ps.append(p + jnp.max(csum))
            return tuple(new_ps)

        ps = lax.fori_loop(0, (SE + 15) // 16, scan_step, (0, 0, 0, 0))

        for g in range(G):
            cbuf[pl.ds(g * 128, 16)] = jnp.full((16,), ps[g], _i32)
            pltpu.sync_copy(stg_e.at[pl.ds(g * CAPF, CAPF)], feid.at[v, g])
            pltpu.sync_copy(stg_s.at[pl.ds(g * CAPF, CAPF)], fsrc.at[v, g])
            pltpu.sync_copy(stg_d.at[pl.ds(g * CAPF, CAPF)], fdl.at[v, g])
        pltpu.sync_copy(cbuf, fcnt.at[v])


def _partition(dst, src):
    mesh = plsc.VectorSubcoreMesh(core_axis_name="c", subcore_axis_name="s")
    k = functools.partial(
        pl.kernel,
        out_type=[
            jax.ShapeDtypeStruct((VS, G, CAPF), _i32),
            jax.ShapeDtypeStruct((VS, G, CAPF), _i32),
            jax.ShapeDtypeStruct((VS, G, CAPF), _i32),
            jax.ShapeDtypeStruct((VS, G * 128), _i32),
        ],
        mesh=mesh,
        compiler_params=pltpu.CompilerParams(needs_layout_passes=False),
        scratch_types=[
            pltpu.VMEM((SE + 16,), _i32),
            pltpu.VMEM((SE + 16,), _i32),
            pltpu.VMEM((G * CAPF,), _i32),
            pltpu.VMEM((G * CAPF,), _i32),
            pltpu.VMEM((G * CAPF,), _i32),
            pltpu.VMEM((G * 128,), _i32),
        ],
    )(_part_body)
    return k(dst, src)


def _segmax_body(with_h, args):
    if with_h:
        (b2, h2, feid, fsrc, fdl, fcnt, out,
         cbuf, ebuf, srbuf, dlbuf, ibb, ibh, bbuf, hbuf, agg,
         semb, semh, seml0, seml1, seml2) = args
    else:
        (b2, feid, fsrc, fdl, fcnt, out,
         cbuf, ebuf, srbuf, dlbuf, ibb, bbuf, agg,
         semb, seml0, seml1) = args
        seml2 = None
    wid = lax.axis_index("s") * NC + lax.axis_index("c")
    g = wid // FB
    fb = wid % FB
    lo = g * PNG
    iota = _iota16()
    zeros = jnp.zeros((16,), _f32)

    def zstep(i, _):
        agg[pl.ds(i * 16, 16)] = zeros
        return 0

    lax.fori_loop(0, PNG + 1, zstep, 0)

    pltpu.sync_copy(fcnt.at[:, pl.ds(g * 128, 128)], cbuf)

    def frag_step(v, _):
        cnt = jnp.max(plsc.load_gather(cbuf, [jnp.full((16,), v, _i32), iota]))
        nch = (cnt + GCH - 1) // GCH

        def chunk_step(ch, _c):
            boff = ch * GCH
            cl0 = pltpu.async_copy(feid.at[v, g, pl.ds(boff, GCH)], ebuf,
                                   seml0)
            cl1 = pltpu.async_copy(fdl.at[v, g, pl.ds(boff, GCH)], dlbuf,
                                   seml1)
            if with_h:
                cl2 = pltpu.async_copy(fsrc.at[v, g, pl.ds(boff, GCH)],
                                       srbuf, seml2)
            cl0.wait()
            if with_h:
                cl2.wait()
            for j in range(GCH // 16):
                e = ebuf[pl.ds(j * 16, 16)]
                ibb[pl.ds(j * 16, 16)] = e * 8 + fb
                if with_h:
                    s = srbuf[pl.ds(j * 16, 16)]
                    ibh[pl.ds(j * 16, 16)] = s * 8 + fb
            cb = pltpu.async_copy(b2.at[ibb], bbuf, semb)
            if with_h:
                chh = pltpu.async_copy(h2.at[ibh], hbuf, semh)
            cl1.wait()
            cb.wait()
            if with_h:
                chh.wait()
            rem = jnp.minimum(cnt - boff, GCH)
            ngr = (rem + 7) // 8

            def hot(k, _h):
                for u in range(8):
                    i = k * 8 + u
                    ispl = jnp.full((16,), i, _i32)
                    av = plsc.load_gather(dlbuf, [ispl]) + iota
                    m = bbuf[i]
                    if with_h:
                        m = m + hbuf[i]
                    a = plsc.load_gather(agg, [av])
                    plsc.store_scatter(agg, [av], jnp.maximum(a, m))
                return 0

            lax.fori_loop(0, ngr, hot, 0)
            return 0

        lax.fori_loop(0, nch, chunk_step, 0)
        return 0

    lax.fori_loop(0, VS, frag_step, 0)

    pltpu.sync_copy(agg.at[pl.ds(0, PNG * 16)], out.at[g, fb])


def _segmax(b2, parts, h2=None):
    feid, fsrc, fdl, fcnt = parts
    with_h = h2 is not None
    mesh = plsc.VectorSubcoreMesh(core_axis_name="c", subcore_axis_name="s")
    scratch = [
        pltpu.VMEM((VS, 128), _i32),
        pltpu.VMEM((GCH,), _i32),
        pltpu.VMEM((GCH,), _i32),
        pltpu.VMEM((GCH,), _i32),
        pltpu.VMEM((GCH,), _i32),
    ]
    if with_h:
        scratch.append(pltpu.VMEM((GCH,), _i32))
    scratch.append(pltpu.VMEM((GCH, 16), _f32))
    if with_h:
        scratch.append(pltpu.VMEM((GCH, 16), _f32))
    scratch.append(pltpu.VMEM(((PNG + 1) * 16,), _f32))
    scratch.append(pltpu.SemaphoreType.DMA)
    if with_h:
        scratch.append(pltpu.SemaphoreType.DMA)
    scratch.append(pltpu.SemaphoreType.DMA)
    scratch.append(pltpu.SemaphoreType.DMA)
    if with_h:
        scratch.append(pltpu.SemaphoreType.DMA)

    def body(*args):
        _segmax_body(with_h, args)

    k = functools.partial(
        pl.kernel,
        out_type=jax.ShapeDtypeStruct((G, FB, PNG * 16), _f32),
        mesh=mesh,
        compiler_params=pltpu.CompilerParams(needs_layout_passes=False,
                                             use_tc_tiling_on_sc=False),
        scratch_types=scratch,
    )(body)
    if with_h:
        out4 = k(b2, h2, feid, fsrc, fdl, fcnt)
    else:
        out4 = k(b2, feid, fsrc, fdl, fcnt)
    out4 = out4.reshape(G, FB, PNG, 16)
    return jnp.transpose(out4, (0, 2, 1, 3)).reshape(N, D)


def kernel(x, edge_index, edge_attr, Wg, We, Wz, Uz, bz, Wr, Ur, br,
           Wh, Uh, bh):
    src = edge_index[0]
    dst = edge_index[1]

    b0, b1 = _edge_mm(edge_attr, We[0], We[1])
    parts = _partition(dst, src)

    agg0 = _segmax(b0.reshape(E * 8, 16), parts)
    hp1 = _gru(x, agg0, Wz[0], Uz[0], bz[0][None], Wr[0], Ur[0], br[0][None],
               Wh[0], Uh[0], bh[0][None], wg=Wg[1])
    agg1 = _segmax(b1.reshape(E * 8, 16), parts, h2=hp1.reshape(N * 8, 16))
    return _gru(x, agg1, Wz[1], Uz[1], bz[1][None], Wr[1], Ur[1], br[1][None],
                Wh[1], Uh[1], bh[1][None])


# speedup vs baseline: 3.0682x; 3.0682x over previous
import functools

import jax
import jax.numpy as jnp
from jax import lax
from jax.experimental import pallas as pl
from jax.experimental.pallas import tpu as pltpu
from jax.experimental.pallas import tpu_sc as plsc

N = 10000
E = 320000
D = 128
ED = 16

NC = 2
NS = 16
NW = NC * NS
VS = 64
SE = E // VS
G = 4
PNG = N // G
FB = 8
GCH = 128
CAPF = 5248

_i32 = jnp.int32
_f32 = jnp.float32


def _iota16():
    return lax.iota(_i32, 16)


def _edge_mm_body(ea_ref, w0_ref, w1_ref, b0_ref, b1_ref):
    ea = ea_ref[...]
    b0_ref[...] = jnp.dot(ea, w0_ref[...], preferred_element_type=_f32)
    b1_ref[...] = jnp.dot(ea, w1_ref[...], preferred_element_type=_f32)


def _edge_mm(ea, we0, we1):
    blk = 2560
    return pl.pallas_call(
        _edge_mm_body,
        grid=(E // blk,),
        in_specs=[
            pl.BlockSpec((blk, ED), lambda i: (i, 0)),
            pl.BlockSpec((ED, D), lambda i: (0, 0)),
            pl.BlockSpec((ED, D), lambda i: (0, 0)),
        ],
        out_specs=[
            pl.BlockSpec((blk, D), lambda i: (i, 0)),
            pl.BlockSpec((blk, D), lambda i: (i, 0)),
        ],
        out_shape=[jax.ShapeDtypeStruct((E, D), _f32)] * 2,
    )(ea, we0, we1)


def _gru_body(emit_hp, x_ref, a_ref, wz, uz, bz, wr, ur, br, wh, uh, bh,
              wg, out_ref):
    x = x_ref[...]
    a = a_ref[...]

    def mm(m, w):
        return jnp.dot(m, w[...], preferred_element_type=_f32)

    z = 1.0 / (1.0 + jnp.exp(-(mm(x, wz) + mm(a, uz) + bz[...])))
    r = 1.0 / (1.0 + jnp.exp(-(mm(x, wr) + mm(a, ur) + br[...])))
    n = jnp.tanh(mm(x, wh) + mm(r * a, uh) + bh[...])
    h = (1.0 - z) * a + z * n
    if emit_hp:
        out_ref[...] = mm(h, wg)
    else:
        out_ref[...] = h


def _gru(x, agg, wz, uz, bz, wr, ur, br, wh, uh, bh, wg=None):
    blk = 2000
    emit_hp = wg is not None
    if wg is None:
        wg = jnp.zeros((D, D), _f32)
    wspec = pl.BlockSpec((D, D), lambda i: (0, 0))
    bspec = pl.BlockSpec((1, D), lambda i: (0, 0))
    return pl.pallas_call(
        functools.partial(_gru_body, emit_hp),
        grid=(N // blk,),
        in_specs=[
            pl.BlockSpec((blk, D), lambda i: (i, 0)),
            pl.BlockSpec((blk, D), lambda i: (i, 0)),
            wspec, wspec, bspec, wspec, wspec, bspec, wspec, wspec, bspec,
            wspec,
        ],
        out_specs=pl.BlockSpec((blk, D), lambda i: (i, 0)),
        out_shape=jax.ShapeDtypeStruct((N, D), _f32),
    )(x, agg, wz, uz, bz, wr, ur, br, wh, uh, bh, wg)


PN2 = 313
CE = 6400
STG = 2 * CE
CAP2 = E + STG + 128


def _part_body(dst_h, src_h, le, ls, ld, lc, dbuf, sbuf, se_, ss_, sd_, cbuf):
    wid = lax.axis_index("s") * NC + lax.axis_index("c")
    iota = _iota16()
    lo = wid * PN2

    fill_d = jnp.full((16,), PN2 << 7, _i32)

    def prefill(i, _):
        k = i * 16 + iota
        se_[pl.ds(i * 16, 16)] = (k * 31) & 0x3FFFF
        ss_[pl.ds(i * 16, 16)] = (k * 13) & 0x1FFF
        sd_[pl.ds(i * 16, 16)] = fill_d
        return 0

    lax.fori_loop(0, STG // 16, prefill, 0)

    def chunk(c, carry):
        p, q = carry
        base = c * CE
        pltpu.sync_copy(dst_h.at[pl.ds(base, CE)], dbuf)
        pltpu.sync_copy(src_h.at[pl.ds(base, CE)], sbuf)

        def step(i, pp):
            d = dbuf[pl.ds(i * 16, 16)]
            s = sbuf[pl.ds(i * 16, 16)]
            msk = (d >= lo) & (d < lo + PN2)
            dl = (d - lo) << 7
            csum = jnp.cumsum(jnp.where(msk, 1, 0).astype(_i32))
            pos = pp + csum - 1
            plsc.store_scatter(se_, [pos], base + i * 16 + iota, mask=msk)
            plsc.store_scatter(ss_, [pos], s, mask=msk)
            plsc.store_scatter(sd_, [pos], dl, mask=msk)
            return pp + jnp.max(csum)

        p = lax.fori_loop(0, CE // 16, step, p)

        def flush(carry2):
            p2, q2 = carry2
            q2 = pl.multiple_of(q2, 128)
            pltpu.sync_copy(se_.at[pl.ds(0, CE)], le.at[wid, 0, pl.ds(q2, CE)])
            pltpu.sync_copy(ss_.at[pl.ds(0, CE)], ls.at[wid, 0, pl.ds(q2, CE)])
            pltpu.sync_copy(sd_.at[pl.ds(0, CE)], ld.at[wid, 0, pl.ds(q2, CE)])

            def shift(j, _):
                se_[pl.ds(j * 16, 16)] = se_[pl.ds(CE + j * 16, 16)]
                ss_[pl.ds(j * 16, 16)] = ss_[pl.ds(CE + j * 16, 16)]
                sd_[pl.ds(j * 16, 16)] = sd_[pl.ds(CE + j * 16, 16)]
                return 0

            lax.fori_loop(0, CE // 16, shift, 0)
            return (p2 - CE, q2 + CE)

        p, q = lax.cond(p >= CE, flush, lambda cc: cc, (p, q))
        return (p, q)

    p, q = lax.fori_loop(0, E // CE, chunk, (0, 0))
    q = pl.multiple_of(q, 128)
    se_[pl.ds(p, 16)] = (lo * 41 + iota * 17) & 0x3FFFF
    ss_[pl.ds(p, 16)] = (lo * 7 + iota * 5) & 0x1FFF
    sd_[pl.ds(p, 16)] = jnp.full((16,), PN2 << 7, _i32)
    pltpu.sync_copy(se_.at[pl.ds(0, CE)], le.at[wid, 0, pl.ds(q, CE)])
    pltpu.sync_copy(ss_.at[pl.ds(0, CE)], ls.at[wid, 0, pl.ds(q, CE)])
    pltpu.sync_copy(sd_.at[pl.ds(0, CE)], ld.at[wid, 0, pl.ds(q, CE)])
    pltpu.sync_copy(se_.at[pl.ds(CE, 128)], le.at[wid, 0, pl.ds(q + CE, 128)])
    pltpu.sync_copy(ss_.at[pl.ds(CE, 128)], ls.at[wid, 0, pl.ds(q + CE, 128)])
    pltpu.sync_copy(sd_.at[pl.ds(CE, 128)], ld.at[wid, 0, pl.ds(q + CE, 128)])
    cbuf[pl.ds(0, 16)] = jnp.full((16,), q + p, _i32)
    pltpu.sync_copy(cbuf, lc.at[wid, 0])


def _partition(dst, src):
    mesh = plsc.VectorSubcoreMesh(core_axis_name="c", subcore_axis_name="s")
    k = functools.partial(
        pl.kernel,
        out_type=[
            jax.ShapeDtypeStruct((NW, 1, CAP2), _i32),
            jax.ShapeDtypeStruct((NW, 1, CAP2), _i32),
            jax.ShapeDtypeStruct((NW, 1, CAP2), _i32),
            jax.ShapeDtypeStruct((NW, 1, 128), _i32),
        ],
        mesh=mesh,
        compiler_params=pltpu.CompilerParams(needs_layout_passes=False),
        scratch_types=[
            pltpu.VMEM((CE,), _i32),
            pltpu.VMEM((CE,), _i32),
            pltpu.VMEM((STG,), _i32),
            pltpu.VMEM((STG,), _i32),
            pltpu.VMEM((STG,), _i32),
            pltpu.VMEM((128,), _i32),
        ],
    )(_part_body)
    return k(dst, src)


def _segmax_body(with_h, args):
    if with_h:
        (bm, hm, le, ls, ld, lc, out,
         cbuf, ebuf, srbuf, dlbuf, bbuf, hbuf, agg,
         semb, semh, seml0, seml1, seml2) = args
    else:
        (bm, le, ls, ld, lc, out,
         cbuf, ebuf, srbuf, dlbuf, bbuf, agg,
         semb, seml0, seml1) = args
    wid = lax.axis_index("s") * NC + lax.axis_index("c")
    iota = _iota16()
    zeros = jnp.zeros((16,), _f32)

    def zstep(i, _):
        agg[pl.ds(i * 16, 16)] = zeros
        return 0

    lax.fori_loop(0, ((PN2 + 1) * 128) // 16, zstep, 0)

    pltpu.sync_copy(lc.at[wid, 0, pl.ds(0, 16)], cbuf)
    cnt = jnp.max(cbuf[pl.ds(0, 16)])
    nch = (cnt + GCH - 1) // GCH

    def chunk_step(ch, _c):
        boff = pl.multiple_of(ch * GCH, 128)
        cl0 = pltpu.async_copy(le.at[wid, 0, pl.ds(boff, GCH)], ebuf, seml0)
        cl1 = pltpu.async_copy(ld.at[wid, 0, pl.ds(boff, GCH)], dlbuf, seml1)
        if with_h:
            cl2 = pltpu.async_copy(ls.at[wid, 0, pl.ds(boff, GCH)], srbuf,
                                   seml2)
        cl0.wait()
        cb = pltpu.async_copy(bm.at[ebuf], bbuf, semb)
        if with_h:
            cl2.wait()
            chh = pltpu.async_copy(hm.at[srbuf], hbuf, semh)
        cl1.wait()
        cb.wait()
        if with_h:
            chh.wait()
        rem = jnp.minimum(cnt - boff, GCH)
        ngr = (rem + 7) // 8

        def hot(k, _h):
            for u in range(8):
                i = k * 8 + u
                ispl = jnp.full((16,), i, _i32)
                av = plsc.load_gather(dlbuf, [ispl]) + iota
                for f in range(8):
                    m = bbuf[i, pl.ds(f * 16, 16)]
                    if with_h:
                        m = m + hbuf[i, pl.ds(f * 16, 16)]
                    avf = av + (f * 16)
                    a = plsc.load_gather(agg, [avf])
                    plsc.store_scatter(agg, [avf], jnp.maximum(a, m))
            return 0

        lax.fori_loop(0, ngr, hot, 0)
        return 0

    lax.fori_loop(0, nch, chunk_step, 0)

    pltpu.sync_copy(agg.at[pl.ds(0, PN2 * 128)], out.at[wid, 0])


def _segmax(bm, parts, hm=None):
    le, ls, ld, lc = parts
    with_h = hm is not None
    mesh = plsc.VectorSubcoreMesh(core_axis_name="c", subcore_axis_name="s")
    scratch = [
        pltpu.VMEM((16,), _i32),
        pltpu.VMEM((GCH,), _i32),
        pltpu.VMEM((GCH,), _i32),
        pltpu.VMEM((GCH,), _i32),
        pltpu.VMEM((GCH, D), _f32),
    ]
    if with_h:
        scratch.append(pltpu.VMEM((GCH, D), _f32))
    scratch.append(pltpu.VMEM(((PN2 + 1) * 128,), _f32))
    scratch.append(pltpu.SemaphoreType.DMA)
    if with_h:
        scratch.append(pltpu.SemaphoreType.DMA)
    scratch.append(pltpu.SemaphoreType.DMA)
    scratch.append(pltpu.SemaphoreType.DMA)
    if with_h:
        scratch.append(pltpu.SemaphoreType.DMA)

    def body(*args):
        _segmax_body(with_h, args)

    k = functools.partial(
        pl.kernel,
        out_type=jax.ShapeDtypeStruct((NW, 1, PN2 * D), _f32),
        mesh=mesh,
        compiler_params=pltpu.CompilerParams(needs_layout_passes=False),
        scratch_types=scratch,
    )(body)
    if with_h:
        out3 = k(bm, hm, le, ls, ld, lc)
    else:
        out3 = k(bm, le, ls, ld, lc)
    return out3.reshape(NW * PN2, D)[:N]


def kernel(x, edge_index, edge_attr, Wg, We, Wz, Uz, bz, Wr, Ur, br,
           Wh, Uh, bh):
    src = edge_index[0]
    dst = edge_index[1]

    b0, b1 = _edge_mm(edge_attr, We[0], We[1])
    parts = _partition(dst, src)

    agg0 = _segmax(b0, parts)
    hp1 = _gru(x, agg0, Wz[0], Uz[0], bz[0][None], Wr[0], Ur[0], br[0][None],
               Wh[0], Uh[0], bh[0][None], wg=Wg[1])
    agg1 = _segmax(b1, parts, hm=hp1)
    return _gru(x, agg1, Wz[1], Uz[1], bz[1][None], Wr[1], Ur[1], br[1][None],
                Wh[1], Uh[1], bh[1][None])


# speedup vs baseline: 3.1967x; 1.0419x over previous
import functools

import jax
import jax.numpy as jnp
from jax import lax
from jax.experimental import pallas as pl
from jax.experimental.pallas import tpu as pltpu
from jax.experimental.pallas import tpu_sc as plsc

N = 10000
E = 320000
D = 128
ED = 16

NC = 2
NS = 16
NW = NC * NS
VS = 64
SE = E // VS
G = 4
PNG = N // G
FB = 8
GCH = 256
CAPF = 5248

_i32 = jnp.int32
_f32 = jnp.float32


def _iota16():
    return lax.iota(_i32, 16)


def _edge_mm_body(ea_ref, w0_ref, w1_ref, b0_ref, b1_ref):
    ea = ea_ref[...]
    b0_ref[...] = jnp.dot(ea, w0_ref[...], preferred_element_type=_f32)
    b1_ref[...] = jnp.dot(ea, w1_ref[...], preferred_element_type=_f32)


def _edge_mm(ea, we0, we1):
    blk = 2560
    return pl.pallas_call(
        _edge_mm_body,
        grid=(E // blk,),
        in_specs=[
            pl.BlockSpec((blk, ED), lambda i: (i, 0)),
            pl.BlockSpec((ED, D), lambda i: (0, 0)),
            pl.BlockSpec((ED, D), lambda i: (0, 0)),
        ],
        out_specs=[
            pl.BlockSpec((blk, D), lambda i: (i, 0)),
            pl.BlockSpec((blk, D), lambda i: (i, 0)),
        ],
        out_shape=[jax.ShapeDtypeStruct((E, D), _f32)] * 2,
    )(ea, we0, we1)


def _gru_body(emit_hp, x_ref, a_ref, wz, uz, bz, wr, ur, br, wh, uh, bh,
              wg, out_ref):
    x = x_ref[...]
    a = a_ref[...]

    def mm(m, w):
        return jnp.dot(m, w[...], preferred_element_type=_f32)

    z = 1.0 / (1.0 + jnp.exp(-(mm(x, wz) + mm(a, uz) + bz[...])))
    r = 1.0 / (1.0 + jnp.exp(-(mm(x, wr) + mm(a, ur) + br[...])))
    n = jnp.tanh(mm(x, wh) + mm(r * a, uh) + bh[...])
    h = (1.0 - z) * a + z * n
    if emit_hp:
        out_ref[...] = mm(h, wg)
    else:
        out_ref[...] = h


def _gru(x, agg, wz, uz, bz, wr, ur, br, wh, uh, bh, wg=None):
    blk = 2000
    emit_hp = wg is not None
    if wg is None:
        wg = jnp.zeros((D, D), _f32)
    wspec = pl.BlockSpec((D, D), lambda i: (0, 0))
    bspec = pl.BlockSpec((1, D), lambda i: (0, 0))
    return pl.pallas_call(
        functools.partial(_gru_body, emit_hp),
        grid=(N // blk,),
        in_specs=[
            pl.BlockSpec((blk, D), lambda i: (i, 0)),
            pl.BlockSpec((blk, D), lambda i: (i, 0)),
            wspec, wspec, bspec, wspec, wspec, bspec, wspec, wspec, bspec,
            wspec,
        ],
        out_specs=pl.BlockSpec((blk, D), lambda i: (i, 0)),
        out_shape=jax.ShapeDtypeStruct((N, D), _f32),
    )(x, agg, wz, uz, bz, wr, ur, br, wh, uh, bh, wg)


PN2 = 313
CE = 6400
STG = 2 * CE
CAP2 = E + STG + 128


def _part_body(dst_h, src_h, le, ls, ld, lc, dbuf, sbuf, se_, ss_, sd_, cbuf):
    wid = lax.axis_index("s") * NC + lax.axis_index("c")
    iota = _iota16()
    lo = wid * PN2

    fill_d = jnp.full((16,), PN2 << 7, _i32)

    def prefill(i, _):
        k = i * 16 + iota
        se_[pl.ds(i * 16, 16)] = (k * 31) & 0x3FFFF
        ss_[pl.ds(i * 16, 16)] = (k * 13) & 0x1FFF
        sd_[pl.ds(i * 16, 16)] = fill_d
        return 0

    lax.fori_loop(0, STG // 16, prefill, 0)

    def chunk(c, carry):
        p, q = carry
        base = c * CE
        pltpu.sync_copy(dst_h.at[pl.ds(base, CE)], dbuf)
        pltpu.sync_copy(src_h.at[pl.ds(base, CE)], sbuf)

        def step(i, pp):
            d = dbuf[pl.ds(i * 16, 16)]
            s = sbuf[pl.ds(i * 16, 16)]
            msk = (d >= lo) & (d < lo + PN2)
            dl = (d - lo) << 7
            csum = jnp.cumsum(jnp.where(msk, 1, 0).astype(_i32))
            pos = pp + csum - 1
            plsc.store_scatter(se_, [pos], base + i * 16 + iota, mask=msk)
            plsc.store_scatter(ss_, [pos], s, mask=msk)
            plsc.store_scatter(sd_, [pos], dl, mask=msk)
            return pp + jnp.max(csum)

        p = lax.fori_loop(0, CE // 16, step, p)

        def flush(carry2):
            p2, q2 = carry2
            q2 = pl.multiple_of(q2, 128)
            pltpu.sync_copy(se_.at[pl.ds(0, CE)], le.at[wid, 0, pl.ds(q2, CE)])
            pltpu.sync_copy(ss_.at[pl.ds(0, CE)], ls.at[wid, 0, pl.ds(q2, CE)])
            pltpu.sync_copy(sd_.at[pl.ds(0, CE)], ld.at[wid, 0, pl.ds(q2, CE)])

            def shift(j, _):
                se_[pl.ds(j * 16, 16)] = se_[pl.ds(CE + j * 16, 16)]
                ss_[pl.ds(j * 16, 16)] = ss_[pl.ds(CE + j * 16, 16)]
                sd_[pl.ds(j * 16, 16)] = sd_[pl.ds(CE + j * 16, 16)]
                return 0

            lax.fori_loop(0, CE // 16, shift, 0)
            return (p2 - CE, q2 + CE)

        p, q = lax.cond(p >= CE, flush, lambda cc: cc, (p, q))
        return (p, q)

    p, q = lax.fori_loop(0, E // CE, chunk, (0, 0))
    q = pl.multiple_of(q, 128)
    se_[pl.ds(p, 16)] = (lo * 41 + iota * 17) & 0x3FFFF
    ss_[pl.ds(p, 16)] = (lo * 7 + iota * 5) & 0x1FFF
    sd_[pl.ds(p, 16)] = jnp.full((16,), PN2 << 7, _i32)
    pltpu.sync_copy(se_.at[pl.ds(0, CE)], le.at[wid, 0, pl.ds(q, CE)])
    pltpu.sync_copy(ss_.at[pl.ds(0, CE)], ls.at[wid, 0, pl.ds(q, CE)])
    pltpu.sync_copy(sd_.at[pl.ds(0, CE)], ld.at[wid, 0, pl.ds(q, CE)])
    pltpu.sync_copy(se_.at[pl.ds(CE, 128)], le.at[wid, 0, pl.ds(q + CE, 128)])
    pltpu.sync_copy(ss_.at[pl.ds(CE, 128)], ls.at[wid, 0, pl.ds(q + CE, 128)])
    pltpu.sync_copy(sd_.at[pl.ds(CE, 128)], ld.at[wid, 0, pl.ds(q + CE, 128)])
    cbuf[pl.ds(0, 16)] = jnp.full((16,), q + p, _i32)
    pltpu.sync_copy(cbuf, lc.at[wid, 0])


def _partition(dst, src):
    mesh = plsc.VectorSubcoreMesh(core_axis_name="c", subcore_axis_name="s")
    k = functools.partial(
        pl.kernel,
        out_type=[
            jax.ShapeDtypeStruct((NW, 1, CAP2), _i32),
            jax.ShapeDtypeStruct((NW, 1, CAP2), _i32),
            jax.ShapeDtypeStruct((NW, 1, CAP2), _i32),
            jax.ShapeDtypeStruct((NW, 1, 128), _i32),
        ],
        mesh=mesh,
        compiler_params=pltpu.CompilerParams(needs_layout_passes=False),
        scratch_types=[
            pltpu.VMEM((CE,), _i32),
            pltpu.VMEM((CE,), _i32),
            pltpu.VMEM((STG,), _i32),
            pltpu.VMEM((STG,), _i32),
            pltpu.VMEM((STG,), _i32),
            pltpu.VMEM((128,), _i32),
        ],
    )(_part_body)
    return k(dst, src)


def _segmax_body(with_h, args):
    if with_h:
        (bm, hm, le, ls, ld, lc, out,
         cbuf, ebuf, srbuf, dlbuf, bbuf, hbuf, agg,
         semb, semh, seml0, seml1, seml2) = args
    else:
        (bm, le, ls, ld, lc, out,
         cbuf, ebuf, srbuf, dlbuf, bbuf, agg,
         semb, seml0, seml1) = args
    wid = lax.axis_index("s") * NC + lax.axis_index("c")
    iota = _iota16()
    zeros = jnp.zeros((16,), _f32)

    def zstep(i, _):
        agg[pl.ds(i * 16, 16)] = zeros
        return 0

    lax.fori_loop(0, ((PN2 + 1) * 128) // 16, zstep, 0)

    pltpu.sync_copy(lc.at[wid, 0, pl.ds(0, 16)], cbuf)
    cnt = jnp.max(cbuf[pl.ds(0, 16)])
    nch = (cnt + GCH - 1) // GCH

    def chunk_step(ch, _c):
        boff = pl.multiple_of(ch * GCH, 128)
        cl0 = pltpu.async_copy(le.at[wid, 0, pl.ds(boff, GCH)], ebuf, seml0)
        cl1 = pltpu.async_copy(ld.at[wid, 0, pl.ds(boff, GCH)], dlbuf, seml1)
        if with_h:
            cl2 = pltpu.async_copy(ls.at[wid, 0, pl.ds(boff, GCH)], srbuf,
                                   seml2)
        cl0.wait()
        cb = pltpu.async_copy(bm.at[ebuf], bbuf, semb)
        if with_h:
            cl2.wait()
            chh = pltpu.async_copy(hm.at[srbuf], hbuf, semh)
        cl1.wait()
        cb.wait()
        if with_h:
            chh.wait()
        rem = jnp.minimum(cnt - boff, GCH)
        ngr = (rem + 7) // 8

        def hot(k, _h):
            for u in range(8):
                i = k * 8 + u
                ispl = jnp.full((16,), i, _i32)
                av = plsc.load_gather(dlbuf, [ispl]) + iota
                for f in range(8):
                    m = bbuf[i, pl.ds(f * 16, 16)]
                    if with_h:
                        m = m + hbuf[i, pl.ds(f * 16, 16)]
                    avf = av + (f * 16)
                    a = plsc.load_gather(agg, [avf])
                    plsc.store_scatter(agg, [avf], jnp.maximum(a, m))
            return 0

        lax.fori_loop(0, ngr, hot, 0)
        return 0

    lax.fori_loop(0, nch, chunk_step, 0)

    pltpu.sync_copy(agg.at[pl.ds(0, PN2 * 128)], out.at[wid, 0])


def _segmax(bm, parts, hm=None):
    le, ls, ld, lc = parts
    with_h = hm is not None
    mesh = plsc.VectorSubcoreMesh(core_axis_name="c", subcore_axis_name="s")
    scratch = [
        pltpu.VMEM((16,), _i32),
        pltpu.VMEM((GCH,), _i32),
        pltpu.VMEM((GCH,), _i32),
        pltpu.VMEM((GCH,), _i32),
        pltpu.VMEM((GCH, D), _f32),
    ]
    if with_h:
        scratch.append(pltpu.VMEM((GCH, D), _f32))
    scratch.append(pltpu.VMEM(((PN2 + 1) * 128,), _f32))
    scratch.append(pltpu.SemaphoreType.DMA)
    if with_h:
        scratch.append(pltpu.SemaphoreType.DMA)
    scratch.append(pltpu.SemaphoreType.DMA)
    scratch.append(pltpu.SemaphoreType.DMA)
    if with_h:
        scratch.append(pltpu.SemaphoreType.DMA)

    def body(*args):
        _segmax_body(with_h, args)

    k = functools.partial(
        pl.kernel,
        out_type=jax.ShapeDtypeStruct((NW, 1, PN2 * D), _f32),
        mesh=mesh,
        compiler_params=pltpu.CompilerParams(needs_layout_passes=False),
        scratch_types=scratch,
    )(body)
    if with_h:
        out3 = k(bm, hm, le, ls, ld, lc)
    else:
        out3 = k(bm, le, ls, ld, lc)
    return out3.reshape(NW * PN2, D)[:N]


def kernel(x, edge_index, edge_attr, Wg, We, Wz, Uz, bz, Wr, Ur, br,
           Wh, Uh, bh):
    src = edge_index[0]
    dst = edge_index[1]

    b0, b1 = _edge_mm(edge_attr, We[0], We[1])
    parts = _partition(dst, src)

    agg0 = _segmax(b0, parts)
    hp1 = _gru(x, agg0, Wz[0], Uz[0], bz[0][None], Wr[0], Ur[0], br[0][None],
               Wh[0], Uh[0], bh[0][None], wg=Wg[1])
    agg1 = _segmax(b1, parts, hm=hp1)
    return _gru(x, agg1, Wz[1], Uz[1], bz[1][None], Wr[1], Ur[1], br[1][None],
                Wh[1], Uh[1], bh[1][None])


# speedup vs baseline: 3.2014x; 1.0015x over previous
import functools

import jax
import jax.numpy as jnp
from jax import lax
from jax.experimental import pallas as pl
from jax.experimental.pallas import tpu as pltpu
from jax.experimental.pallas import tpu_sc as plsc

N = 10000
E = 320000
D = 128
ED = 16

NC = 2
NS = 16
NW = NC * NS
GCH = 256

_i32 = jnp.int32
_f32 = jnp.float32


def _iota16():
    return lax.iota(_i32, 16)


def _edge_mm_body(ea_ref, w0_ref, w1_ref, b0_ref, b1_ref):
    ea = ea_ref[...]
    b0_ref[...] = jnp.dot(ea, w0_ref[...], preferred_element_type=_f32)
    b1_ref[...] = jnp.dot(ea, w1_ref[...], preferred_element_type=_f32)


def _edge_mm(ea, we0, we1):
    blk = 2560
    return pl.pallas_call(
        _edge_mm_body,
        grid=(E // blk,),
        in_specs=[
            pl.BlockSpec((blk, ED), lambda i: (i, 0)),
            pl.BlockSpec((ED, D), lambda i: (0, 0)),
            pl.BlockSpec((ED, D), lambda i: (0, 0)),
        ],
        out_specs=[
            pl.BlockSpec((blk, D), lambda i: (i, 0)),
            pl.BlockSpec((blk, D), lambda i: (i, 0)),
        ],
        out_shape=[jax.ShapeDtypeStruct((E, D), _f32)] * 2,
    )(ea, we0, we1)


def _gru_body(emit_hp, x_ref, a_ref, wz, uz, bz, wr, ur, br, wh, uh, bh,
              wg, out_ref):
    x = x_ref[...]
    a = a_ref[...]

    def mm(m, w):
        return jnp.dot(m, w[...], preferred_element_type=_f32)

    z = 1.0 / (1.0 + jnp.exp(-(mm(x, wz) + mm(a, uz) + bz[...])))
    r = 1.0 / (1.0 + jnp.exp(-(mm(x, wr) + mm(a, ur) + br[...])))
    n = jnp.tanh(mm(x, wh) + mm(r * a, uh) + bh[...])
    h = (1.0 - z) * a + z * n
    if emit_hp:
        out_ref[...] = mm(h, wg)
    else:
        out_ref[...] = h


def _gru(x, agg, wz, uz, bz, wr, ur, br, wh, uh, bh, wg=None):
    blk = 2000
    emit_hp = wg is not None
    if wg is None:
        wg = jnp.zeros((D, D), _f32)
    wspec = pl.BlockSpec((D, D), lambda i: (0, 0))
    bspec = pl.BlockSpec((1, D), lambda i: (0, 0))
    return pl.pallas_call(
        functools.partial(_gru_body, emit_hp),
        grid=(N // blk,),
        in_specs=[
            pl.BlockSpec((blk, D), lambda i: (i, 0)),
            pl.BlockSpec((blk, D), lambda i: (i, 0)),
            wspec, wspec, bspec, wspec, wspec, bspec, wspec, wspec, bspec,
            wspec,
        ],
        out_specs=pl.BlockSpec((blk, D), lambda i: (i, 0)),
        out_shape=jax.ShapeDtypeStruct((N, D), _f32),
    )(x, agg, wz, uz, bz, wr, ur, br, wh, uh, bh, wg)


PN2 = 313
CE = 6400
STG = 2 * CE
CAP2 = E + STG + 128


def _part_body(dst_h, src_h, le, ls, ld, lc, dbuf, sbuf, se_, ss_, sd_, cbuf):
    wid = lax.axis_index("s") * NC + lax.axis_index("c")
    iota = _iota16()
    lo = wid * PN2

    fill_d = jnp.full((16,), PN2 << 7, _i32)

    def prefill(i, _):
        k = i * 16 + iota
        se_[pl.ds(i * 16, 16)] = (k * 31) & 0x3FFFF
        ss_[pl.ds(i * 16, 16)] = (k * 13) & 0x1FFF
        sd_[pl.ds(i * 16, 16)] = fill_d
        return 0

    lax.fori_loop(0, STG // 16, prefill, 0)

    def chunk(c, carry):
        p, q = carry
        base = c * CE
        pltpu.sync_copy(dst_h.at[pl.ds(base, CE)], dbuf)
        pltpu.sync_copy(src_h.at[pl.ds(base, CE)], sbuf)

        def step(i, pp):
            d = dbuf[pl.ds(i * 16, 16)]
            s = sbuf[pl.ds(i * 16, 16)]
            msk = (d >= lo) & (d < lo + PN2)
            dl = (d - lo) << 7
            csum = jnp.cumsum(jnp.where(msk, 1, 0).astype(_i32))
            pos = pp + csum - 1
            plsc.store_scatter(se_, [pos], base + i * 16 + iota, mask=msk)
            plsc.store_scatter(ss_, [pos], s, mask=msk)
            plsc.store_scatter(sd_, [pos], dl, mask=msk)
            return pp + jnp.max(csum)

        p = lax.fori_loop(0, CE // 16, step, p)

        def flush(carry2):
            p2, q2 = carry2
            q2 = pl.multiple_of(q2, 128)
            pltpu.sync_copy(se_.at[pl.ds(0, CE)], le.at[wid, 0, pl.ds(q2, CE)])
            pltpu.sync_copy(ss_.at[pl.ds(0, CE)], ls.at[wid, 0, pl.ds(q2, CE)])
            pltpu.sync_copy(sd_.at[pl.ds(0, CE)], ld.at[wid, 0, pl.ds(q2, CE)])

            def shift(j, _):
                se_[pl.ds(j * 16, 16)] = se_[pl.ds(CE + j * 16, 16)]
                ss_[pl.ds(j * 16, 16)] = ss_[pl.ds(CE + j * 16, 16)]
                sd_[pl.ds(j * 16, 16)] = sd_[pl.ds(CE + j * 16, 16)]
                return 0

            lax.fori_loop(0, CE // 16, shift, 0)
            return (p2 - CE, q2 + CE)

        p, q = lax.cond(p >= CE, flush, lambda cc: cc, (p, q))
        return (p, q)

    p, q = lax.fori_loop(0, E // CE, chunk, (0, 0))
    q = pl.multiple_of(q, 128)
    se_[pl.ds(p, 16)] = (lo * 41 + iota * 17) & 0x3FFFF
    ss_[pl.ds(p, 16)] = (lo * 7 + iota * 5) & 0x1FFF
    sd_[pl.ds(p, 16)] = jnp.full((16,), PN2 << 7, _i32)
    pltpu.sync_copy(se_.at[pl.ds(0, CE)], le.at[wid, 0, pl.ds(q, CE)])
    pltpu.sync_copy(ss_.at[pl.ds(0, CE)], ls.at[wid, 0, pl.ds(q, CE)])
    pltpu.sync_copy(sd_.at[pl.ds(0, CE)], ld.at[wid, 0, pl.ds(q, CE)])
    pltpu.sync_copy(se_.at[pl.ds(CE, 128)], le.at[wid, 0, pl.ds(q + CE, 128)])
    pltpu.sync_copy(ss_.at[pl.ds(CE, 128)], ls.at[wid, 0, pl.ds(q + CE, 128)])
    pltpu.sync_copy(sd_.at[pl.ds(CE, 128)], ld.at[wid, 0, pl.ds(q + CE, 128)])
    cbuf[pl.ds(0, 16)] = jnp.full((16,), q + p, _i32)
    pltpu.sync_copy(cbuf, lc.at[wid, 0])


def _partition(dst, src):
    mesh = plsc.VectorSubcoreMesh(core_axis_name="c", subcore_axis_name="s")
    k = functools.partial(
        pl.kernel,
        out_type=[
            jax.ShapeDtypeStruct((NW, 1, CAP2), _i32),
            jax.ShapeDtypeStruct((NW, 1, CAP2), _i32),
            jax.ShapeDtypeStruct((NW, 1, CAP2), _i32),
            jax.ShapeDtypeStruct((NW, 1, 128), _i32),
        ],
        mesh=mesh,
        compiler_params=pltpu.CompilerParams(needs_layout_passes=False),
        scratch_types=[
            pltpu.VMEM((CE,), _i32),
            pltpu.VMEM((CE,), _i32),
            pltpu.VMEM((STG,), _i32),
            pltpu.VMEM((STG,), _i32),
            pltpu.VMEM((STG,), _i32),
            pltpu.VMEM((128,), _i32),
        ],
    )(_part_body)
    return k(dst, src)


def _segmax_body(with_h, args):
    if with_h:
        (bm, hm, le, ls, ld, lc, out,
         cbuf, ebuf, srbuf, dlbuf, bbuf, hbuf, agg,
         semb, semh, seml0, seml1, seml2) = args
    else:
        (bm, le, ls, ld, lc, out,
         cbuf, ebuf, srbuf, dlbuf, bbuf, agg,
         semb, seml0, seml1) = args
    wid = lax.axis_index("s") * NC + lax.axis_index("c")
    iota = _iota16()
    zeros = jnp.zeros((16,), _f32)

    def zstep(i, _):
        agg[pl.ds(i * 16, 16)] = zeros
        return 0

    lax.fori_loop(0, ((PN2 + 1) * 128) // 16, zstep, 0)

    pltpu.sync_copy(lc.at[wid, 0, pl.ds(0, 16)], cbuf)
    cnt = jnp.max(cbuf[pl.ds(0, 16)])
    nch = (cnt + GCH - 1) // GCH

    def chunk_step(ch, _c):
        boff = pl.multiple_of(ch * GCH, 128)
        cl0 = pltpu.async_copy(le.at[wid, 0, pl.ds(boff, GCH)], ebuf, seml0)
        cl1 = pltpu.async_copy(ld.at[wid, 0, pl.ds(boff, GCH)], dlbuf, seml1)
        if with_h:
            cl2 = pltpu.async_copy(ls.at[wid, 0, pl.ds(boff, GCH)], srbuf,
                                   seml2)
        cl0.wait()
        cb = pltpu.async_copy(bm.at[ebuf], bbuf, semb)
        if with_h:
            cl2.wait()
            chh = pltpu.async_copy(hm.at[srbuf], hbuf, semh)
        cl1.wait()
        cb.wait()
        if with_h:
            chh.wait()
        rem = jnp.minimum(cnt - boff, GCH)
        ngr = (rem + 7) // 8

        def hot(k, _h):
            for u in range(8):
                i = k * 8 + u
                ispl = jnp.full((16,), i, _i32)
                av = plsc.load_gather(dlbuf, [ispl]) + iota
                for f in range(8):
                    m = bbuf[i, pl.ds(f * 16, 16)]
                    if with_h:
                        m = m + hbuf[i, pl.ds(f * 16, 16)]
                    avf = av + (f * 16)
                    a = plsc.load_gather(agg, [avf])
                    plsc.store_scatter(agg, [avf], jnp.maximum(a, m))
            return 0

        lax.fori_loop(0, ngr, hot, 0)
        return 0

    lax.fori_loop(0, nch, chunk_step, 0)

    pltpu.sync_copy(agg.at[pl.ds(0, PN2 * 128)], out.at[wid, 0])


def _segmax(bm, parts, hm=None):
    le, ls, ld, lc = parts
    with_h = hm is not None
    mesh = plsc.VectorSubcoreMesh(core_axis_name="c", subcore_axis_name="s")
    scratch = [
        pltpu.VMEM((16,), _i32),
        pltpu.VMEM((GCH,), _i32),
        pltpu.VMEM((GCH,), _i32),
        pltpu.VMEM((GCH,), _i32),
        pltpu.VMEM((GCH, D), _f32),
    ]
    if with_h:
        scratch.append(pltpu.VMEM((GCH, D), _f32))
    scratch.append(pltpu.VMEM(((PN2 + 1) * 128,), _f32))
    scratch.append(pltpu.SemaphoreType.DMA)
    if with_h:
        scratch.append(pltpu.SemaphoreType.DMA)
    scratch.append(pltpu.SemaphoreType.DMA)
    scratch.append(pltpu.SemaphoreType.DMA)
    if with_h:
        scratch.append(pltpu.SemaphoreType.DMA)

    def body(*args):
        _segmax_body(with_h, args)

    k = functools.partial(
        pl.kernel,
        out_type=jax.ShapeDtypeStruct((NW, 1, PN2 * D), _f32),
        mesh=mesh,
        compiler_params=pltpu.CompilerParams(needs_layout_passes=False),
        scratch_types=scratch,
    )(body)
    if with_h:
        out3 = k(bm, hm, le, ls, ld, lc)
    else:
        out3 = k(bm, le, ls, ld, lc)
    return out3.reshape(NW * PN2, D)[:N]


def kernel(x, edge_index, edge_attr, Wg, We, Wz, Uz, bz, Wr, Ur, br,
           Wh, Uh, bh):
    src = edge_index[0]
    dst = edge_index[1]

    b0, b1 = _edge_mm(edge_attr, We[0], We[1])
    parts = _partition(dst, src)

    agg0 = _segmax(b0, parts)
    hp1 = _gru(x, agg0, Wz[0], Uz[0], bz[0][None], Wr[0], Ur[0], br[0][None],
               Wh[0], Uh[0], bh[0][None], wg=Wg[1])
    agg1 = _segmax(b1, parts, hm=hp1)
    return _gru(x, agg1, Wz[1], Uz[1], bz[1][None], Wr[1], Ur[1], br[1][None],
                Wh[1], Uh[1], bh[1][None])
